# baseline (speedup 1.0000x reference)
import functools

import jax
import jax.numpy as jnp
from jax import lax
from jax.experimental import pallas as pl
from jax.experimental.pallas import tpu as pltpu
from jax.experimental.pallas import tpu_sc as plsc

B = 16384
D = 32
NC = 2
NS = 16
L = 16
NW = NC * NS
BPW = B // NW
CH = 128
NCH = BPW // CH


def _sc_partials(u_idx, p_idx, n_idx, user_table, item_table):
    mesh = plsc.VectorSubcoreMesh(core_axis_name="c", subcore_axis_name="s")

    @functools.partial(
        pl.kernel,
        mesh=mesh,
        out_type=jax.ShapeDtypeStruct((B * L,), jnp.float32),
        compiler_params=pltpu.CompilerParams(use_tc_tiling_on_sc=False),
        scratch_types=[
            pltpu.VMEM((BPW,), jnp.int32),
            pltpu.VMEM((BPW,), jnp.int32),
            pltpu.VMEM((BPW,), jnp.int32),
            pltpu.VMEM((BPW, D), jnp.float32),
            pltpu.VMEM((BPW, D), jnp.float32),
            pltpu.VMEM((BPW, D), jnp.float32),
            pltpu.VMEM((BPW * L,), jnp.float32),
            pltpu.SemaphoreType.DMA,
        ],
    )
    def k(u_idx_hbm, p_idx_hbm, n_idx_hbm, ut_hbm, it_hbm, out_hbm,
          iu, ip, ineg, ru, rp, rn, hp, sem):
        wid = lax.axis_index("s") * NC + lax.axis_index("c")
        base = wid * BPW
        pltpu.sync_copy(u_idx_hbm.at[pl.ds(base, BPW)], iu)
        pltpu.sync_copy(p_idx_hbm.at[pl.ds(base, BPW)], ip)
        pltpu.sync_copy(n_idx_hbm.at[pl.ds(base, BPW)], ineg)

        handles = []
        for t in range(NCH):
            sl = pl.ds(t * CH, CH)
            handles.append(pltpu.async_copy(ut_hbm.at[iu.at[sl]], ru.at[sl], sem))
            handles.append(pltpu.async_copy(it_hbm.at[ip.at[sl]], rp.at[sl], sem))
            handles.append(pltpu.async_copy(it_hbm.at[ineg.at[sl]], rn.at[sl], sem))
        for h in handles:
            h.wait()

        def rowbody(i, carry):
            u0 = ru[i, pl.ds(0, L)]
            u1 = ru[i, pl.ds(L, L)]
            p0 = rp[i, pl.ds(0, L)]
            p1 = rp[i, pl.ds(L, L)]
            n0 = rn[i, pl.ds(0, L)]
            n1 = rn[i, pl.ds(L, L)]
            hp[pl.ds(i * L, L)] = u0 * (p0 - n0) + u1 * (p1 - n1)
            return carry

        lax.fori_loop(0, BPW, rowbody, 0)

        pltpu.sync_copy(hp, out_hbm.at[pl.ds(base * L, BPW * L)])

    return k(u_idx, p_idx, n_idx, user_table, item_table)


N = 1000000
TW = 4096
Q = TW // 4
NG = (N + TW - 1) // TW
NP = NG * TW


def _tc_relayout(t_t):
    def body(i_ref, oi_ref):
        y = i_ref[...]
        oi_ref[...] = jnp.concatenate(
            [y[:, a * Q:(a + 1) * Q].T for a in range(4)], axis=1)

    return pl.pallas_call(
        body,
        grid=(NG,),
        in_specs=[pl.BlockSpec((D, TW), lambda g: (0, g))],
        out_specs=pl.BlockSpec((Q, 128), lambda g: (g, 0)),
        out_shape=jax.ShapeDtypeStruct((NG * Q, 128), jnp.float32),
    )(t_t)


def _packed_idx(r):
    return (r >> 12) * TW + ((r & (Q - 1)) << 2) + ((r >> 10) & 3)


def _tc_loss(x2d):
    def body(x_ref, o_ref):
        x = x_ref[...]
        k = lax.iota(jnp.int32, 128)
        sel = (k[:, None] // L == lax.iota(jnp.int32, 8)[None, :])
        m = sel.astype(jnp.float32)
        d = jnp.dot(x, m, preferred_element_type=jnp.float32)
        sp = jnp.maximum(-d, 0.0) + jnp.log(1.0 + jnp.exp(-jnp.abs(d)))
        o_ref[0, 0] = jnp.sum(sp) * (1.0 / B)

    return pl.pallas_call(
        body,
        out_shape=jax.ShapeDtypeStruct((1, 1), jnp.float32),
        in_specs=[pl.BlockSpec(memory_space=pltpu.VMEM)],
        out_specs=pl.BlockSpec(memory_space=pltpu.SMEM),
    )(x2d)


def kernel(user_indices, pos_item_indices, neg_item_indices, user_table, item_table):
    u_idx = user_indices.astype(jnp.int32)
    p_idx = pos_item_indices.astype(jnp.int32)
    n_idx = neg_item_indices.astype(jnp.int32)
    it_pack = _tc_relayout(item_table.T)
    partials = _sc_partials(
        u_idx, _packed_idx(p_idx), _packed_idx(n_idx),
        user_table, it_pack.reshape(NP, D))
    loss = _tc_loss(partials.reshape(B * L // 128, 128))
    return loss[0, 0]


# speedup vs baseline: 1.2947x; 1.2947x over previous
import functools

import jax
import jax.numpy as jnp
from jax import lax
from jax.experimental import pallas as pl
from jax.experimental.pallas import tpu as pltpu
from jax.experimental.pallas import tpu_sc as plsc

B = 16384
D = 32
NC = 2
NS = 16
L = 16
NW = NC * NS
BPW = B // NW
CH = 128
NCH = BPW // CH


def _sc_partials(u_idx, p_idx, n_idx, user_table, item_table):
    mesh = plsc.VectorSubcoreMesh(core_axis_name="c", subcore_axis_name="s")

    @functools.partial(
        pl.kernel,
        mesh=mesh,
        out_type=jax.ShapeDtypeStruct((B * L,), jnp.float32),
        compiler_params=pltpu.CompilerParams(use_tc_tiling_on_sc=False),
        scratch_types=[
            pltpu.VMEM((BPW,), jnp.int32),
            pltpu.VMEM((BPW,), jnp.int32),
            pltpu.VMEM((BPW,), jnp.int32),
            pltpu.VMEM((BPW, D), jnp.float32),
            pltpu.VMEM((BPW, D), jnp.float32),
            pltpu.VMEM((BPW, D), jnp.float32),
            pltpu.VMEM((BPW * L,), jnp.float32),
            pltpu.SemaphoreType.DMA,
        ],
    )
    def k(u_idx_hbm, p_idx_hbm, n_idx_hbm, ut_hbm, it_hbm, out_hbm,
          iu, ip, ineg, ru, rp, rn, hp, sem):
        wid = lax.axis_index("s") * NC + lax.axis_index("c")
        base = wid * BPW
        pltpu.sync_copy(u_idx_hbm.at[pl.ds(base, BPW)], iu)
        pltpu.sync_copy(p_idx_hbm.at[pl.ds(base, BPW)], ip)
        pltpu.sync_copy(n_idx_hbm.at[pl.ds(base, BPW)], ineg)

        handles = []
        for t in range(NCH):
            sl = pl.ds(t * CH, CH)
            handles.append(pltpu.async_copy(ut_hbm.at[iu.at[sl]], ru.at[sl], sem))
            handles.append(pltpu.async_copy(it_hbm.at[ip.at[sl]], rp.at[sl], sem))
            handles.append(pltpu.async_copy(it_hbm.at[ineg.at[sl]], rn.at[sl], sem))
        for h in handles:
            h.wait()

        def rowbody(i, carry):
            u0 = ru[i, pl.ds(0, L)]
            u1 = ru[i, pl.ds(L, L)]
            p0 = rp[i, pl.ds(0, L)]
            p1 = rp[i, pl.ds(L, L)]
            n0 = rn[i, pl.ds(0, L)]
            n1 = rn[i, pl.ds(L, L)]
            hp[pl.ds(i * L, L)] = u0 * (p0 - n0) + u1 * (p1 - n1)
            return carry

        lax.fori_loop(0, BPW, rowbody, 0)

        pltpu.sync_copy(hp, out_hbm.at[pl.ds(base * L, BPW * L)])

    return k(u_idx, p_idx, n_idx, user_table, item_table)


N = 1000000
TW = 4096
Q = TW // 4
NG = (N + TW - 1) // TW
NP = NG * TW


def _tc_relayout(ut_t, it_t):
    def body(u_ref, i_ref, ou_ref, oi_ref):
        x = u_ref[...]
        ou_ref[...] = jnp.concatenate(
            [x[:, a * Q:(a + 1) * Q].T for a in range(4)], axis=1)
        y = i_ref[...]
        oi_ref[...] = jnp.concatenate(
            [y[:, a * Q:(a + 1) * Q].T for a in range(4)], axis=1)

    return pl.pallas_call(
        body,
        grid=(NG,),
        in_specs=[
            pl.BlockSpec((D, TW), lambda g: (0, g)),
            pl.BlockSpec((D, TW), lambda g: (0, g)),
        ],
        out_specs=[
            pl.BlockSpec((Q, 128), lambda g: (g, 0)),
            pl.BlockSpec((Q, 128), lambda g: (g, 0)),
        ],
        out_shape=[
            jax.ShapeDtypeStruct((NG * Q, 128), jnp.float32),
            jax.ShapeDtypeStruct((NG * Q, 128), jnp.float32),
        ],
    )(ut_t, it_t)


def _packed_idx(r):
    return (r >> 12) * TW + ((r & (Q - 1)) << 2) + ((r >> 10) & 3)


def _tc_loss(x2d):
    def body(x_ref, o_ref):
        x = x_ref[...]
        k = lax.iota(jnp.int32, 128)
        sel = (k[:, None] // L == lax.iota(jnp.int32, 8)[None, :])
        m = sel.astype(jnp.float32)
        d = jnp.dot(x, m, preferred_element_type=jnp.float32)
        sp = jnp.maximum(-d, 0.0) + jnp.log(1.0 + jnp.exp(-jnp.abs(d)))
        o_ref[0, 0] = jnp.sum(sp) * (1.0 / B)

    return pl.pallas_call(
        body,
        out_shape=jax.ShapeDtypeStruct((1, 1), jnp.float32),
        in_specs=[pl.BlockSpec(memory_space=pltpu.VMEM)],
        out_specs=pl.BlockSpec(memory_space=pltpu.SMEM),
    )(x2d)


def kernel(user_indices, pos_item_indices, neg_item_indices, user_table, item_table):
    u_idx = user_indices.astype(jnp.int32)
    p_idx = pos_item_indices.astype(jnp.int32)
    n_idx = neg_item_indices.astype(jnp.int32)
    ut_pack, it_pack = _tc_relayout(user_table.T, item_table.T)
    partials = _sc_partials(
        _packed_idx(u_idx), _packed_idx(p_idx), _packed_idx(n_idx),
        ut_pack.reshape(NP, D), it_pack.reshape(NP, D))
    loss = _tc_loss(partials.reshape(B * L // 128, 128))
    return loss[0, 0]


# speedup vs baseline: 1.3250x; 1.0234x over previous
import functools

import jax
import jax.numpy as jnp
from jax import lax
from jax.experimental import pallas as pl
from jax.experimental.pallas import tpu as pltpu
from jax.experimental.pallas import tpu_sc as plsc

B = 16384
D = 32
NC = 2
NS = 16
L = 16
NW = NC * NS
BPW = B // NW
CH = 128
NCH = BPW // CH


def _sc_partials(u_idx, p_idx, n_idx, user_table, item_table):
    mesh = plsc.VectorSubcoreMesh(core_axis_name="c", subcore_axis_name="s")

    @functools.partial(
        pl.kernel,
        mesh=mesh,
        out_type=jax.ShapeDtypeStruct((B * L,), jnp.float32),
        compiler_params=pltpu.CompilerParams(use_tc_tiling_on_sc=False),
        scratch_types=[
            pltpu.VMEM((BPW,), jnp.int32),
            pltpu.VMEM((BPW,), jnp.int32),
            pltpu.VMEM((BPW,), jnp.int32),
            pltpu.VMEM((BPW, D), jnp.float32),
            pltpu.VMEM((BPW, D), jnp.float32),
            pltpu.VMEM((BPW, D), jnp.float32),
            pltpu.VMEM((BPW * L,), jnp.float32),
            pltpu.SemaphoreType.DMA,
        ],
    )
    def k(u_idx_hbm, p_idx_hbm, n_idx_hbm, ut_hbm, it_hbm, out_hbm,
          iu, ip, ineg, ru, rp, rn, hp, sem):
        wid = lax.axis_index("s") * NC + lax.axis_index("c")
        base = wid * BPW
        pltpu.sync_copy(u_idx_hbm.at[pl.ds(base, BPW)], iu)
        pltpu.sync_copy(p_idx_hbm.at[pl.ds(base, BPW)], ip)
        pltpu.sync_copy(n_idx_hbm.at[pl.ds(base, BPW)], ineg)

        handles = []
        for t in range(NCH):
            sl = pl.ds(t * CH, CH)
            handles.append(pltpu.async_copy(ut_hbm.at[iu.at[sl]], ru.at[sl], sem))
            handles.append(pltpu.async_copy(it_hbm.at[ip.at[sl]], rp.at[sl], sem))
            handles.append(pltpu.async_copy(it_hbm.at[ineg.at[sl]], rn.at[sl], sem))
        for h in handles:
            h.wait()

        def rowbody(i, carry):
            u0 = ru[i, pl.ds(0, L)]
            u1 = ru[i, pl.ds(L, L)]
            p0 = rp[i, pl.ds(0, L)]
            p1 = rp[i, pl.ds(L, L)]
            n0 = rn[i, pl.ds(0, L)]
            n1 = rn[i, pl.ds(L, L)]
            hp[pl.ds(i * L, L)] = u0 * (p0 - n0) + u1 * (p1 - n1)
            return carry

        lax.fori_loop(0, BPW, rowbody, 0)

        pltpu.sync_copy(hp, out_hbm.at[pl.ds(base * L, BPW * L)])

    return k(u_idx, p_idx, n_idx, user_table, item_table)


N = 1000000
TW = 8192
Q = TW // 4
NG = (N + TW - 1) // TW
NP = NG * TW


def _tc_relayout(ut_t, it_t):
    def body(u_ref, i_ref, ou_ref, oi_ref):
        x = u_ref[...]
        ou_ref[...] = jnp.concatenate(
            [x[:, a * Q:(a + 1) * Q].T for a in range(4)], axis=1)
        y = i_ref[...]
        oi_ref[...] = jnp.concatenate(
            [y[:, a * Q:(a + 1) * Q].T for a in range(4)], axis=1)

    return pl.pallas_call(
        body,
        grid=(NG,),
        in_specs=[
            pl.BlockSpec((D, TW), lambda g: (0, g)),
            pl.BlockSpec((D, TW), lambda g: (0, g)),
        ],
        out_specs=[
            pl.BlockSpec((Q, 128), lambda g: (g, 0)),
            pl.BlockSpec((Q, 128), lambda g: (g, 0)),
        ],
        out_shape=[
            jax.ShapeDtypeStruct((NG * Q, 128), jnp.float32),
            jax.ShapeDtypeStruct((NG * Q, 128), jnp.float32),
        ],
    )(ut_t, it_t)


_TW_SHIFT = TW.bit_length() - 1
_Q_SHIFT = Q.bit_length() - 1


def _packed_idx(r):
    return ((r >> _TW_SHIFT) * TW + ((r & (Q - 1)) << 2)
            + ((r >> _Q_SHIFT) & 3))


def _tc_loss(x2d):
    def body(x_ref, o_ref):
        x = x_ref[...]
        k = lax.iota(jnp.int32, 128)
        sel = (k[:, None] // L == lax.iota(jnp.int32, 8)[None, :])
        m = sel.astype(jnp.float32)
        d = jnp.dot(x, m, preferred_element_type=jnp.float32)
        sp = jnp.maximum(-d, 0.0) + jnp.log(1.0 + jnp.exp(-jnp.abs(d)))
        o_ref[0, 0] = jnp.sum(sp) * (1.0 / B)

    return pl.pallas_call(
        body,
        out_shape=jax.ShapeDtypeStruct((1, 1), jnp.float32),
        in_specs=[pl.BlockSpec(memory_space=pltpu.VMEM)],
        out_specs=pl.BlockSpec(memory_space=pltpu.SMEM),
    )(x2d)


def kernel(user_indices, pos_item_indices, neg_item_indices, user_table, item_table):
    u_idx = user_indices.astype(jnp.int32)
    p_idx = pos_item_indices.astype(jnp.int32)
    n_idx = neg_item_indices.astype(jnp.int32)
    ut_pack, it_pack = _tc_relayout(user_table.T, item_table.T)
    partials = _sc_partials(
        _packed_idx(u_idx), _packed_idx(p_idx), _packed_idx(n_idx),
        ut_pack.reshape(NP, D), it_pack.reshape(NP, D))
    loss = _tc_loss(partials.reshape(B * L // 128, 128))
    return loss[0, 0]


# speedup vs baseline: 1.3287x; 1.0028x over previous
import functools

import jax
import jax.numpy as jnp
from jax import lax
from jax.experimental import pallas as pl
from jax.experimental.pallas import tpu as pltpu
from jax.experimental.pallas import tpu_sc as plsc

B = 16384
D = 32
NC = 2
NS = 16
L = 16
NW = NC * NS
BPW = B // NW
CH = 128
NCH = BPW // CH


def _sc_partials(u_idx, p_idx, n_idx, user_table, item_table):
    mesh = plsc.VectorSubcoreMesh(core_axis_name="c", subcore_axis_name="s")

    @functools.partial(
        pl.kernel,
        mesh=mesh,
        out_type=jax.ShapeDtypeStruct((B * L,), jnp.float32),
        compiler_params=pltpu.CompilerParams(use_tc_tiling_on_sc=False),
        scratch_types=[
            pltpu.VMEM((BPW,), jnp.int32),
            pltpu.VMEM((BPW,), jnp.int32),
            pltpu.VMEM((BPW,), jnp.int32),
            pltpu.VMEM((BPW, D), jnp.float32),
            pltpu.VMEM((BPW, D), jnp.float32),
            pltpu.VMEM((BPW, D), jnp.float32),
            pltpu.VMEM((BPW * L,), jnp.float32),
            pltpu.SemaphoreType.DMA,
        ],
    )
    def k(u_idx_hbm, p_idx_hbm, n_idx_hbm, ut_hbm, it_hbm, out_hbm,
          iu, ip, ineg, ru, rp, rn, hp, sem):
        wid = lax.axis_index("s") * NC + lax.axis_index("c")
        base = wid * BPW
        pltpu.sync_copy(u_idx_hbm.at[pl.ds(base, BPW)], iu)
        pltpu.sync_copy(p_idx_hbm.at[pl.ds(base, BPW)], ip)
        pltpu.sync_copy(n_idx_hbm.at[pl.ds(base, BPW)], ineg)

        handles = []
        for t in range(NCH):
            sl = pl.ds(t * CH, CH)
            handles.append(pltpu.async_copy(ut_hbm.at[iu.at[sl]], ru.at[sl], sem))
            handles.append(pltpu.async_copy(it_hbm.at[ip.at[sl]], rp.at[sl], sem))
            handles.append(pltpu.async_copy(it_hbm.at[ineg.at[sl]], rn.at[sl], sem))
        for h in handles:
            h.wait()

        def rowbody(i, carry):
            u0 = ru[i, pl.ds(0, L)]
            u1 = ru[i, pl.ds(L, L)]
            p0 = rp[i, pl.ds(0, L)]
            p1 = rp[i, pl.ds(L, L)]
            n0 = rn[i, pl.ds(0, L)]
            n1 = rn[i, pl.ds(L, L)]
            hp[pl.ds(i * L, L)] = u0 * (p0 - n0) + u1 * (p1 - n1)
            return carry

        lax.fori_loop(0, BPW, rowbody, 0)

        pltpu.sync_copy(hp, out_hbm.at[pl.ds(base * L, BPW * L)])

    return k(u_idx, p_idx, n_idx, user_table, item_table)


N = 1000000
TW = 16384
Q = TW // 4
NG = (N + TW - 1) // TW
NP = NG * TW


def _tc_relayout(ut_t, it_t):
    def body(u_ref, i_ref, ou_ref, oi_ref):
        x = u_ref[...]
        ou_ref[...] = jnp.concatenate(
            [x[:, a * Q:(a + 1) * Q].T for a in range(4)], axis=1)
        y = i_ref[...]
        oi_ref[...] = jnp.concatenate(
            [y[:, a * Q:(a + 1) * Q].T for a in range(4)], axis=1)

    return pl.pallas_call(
        body,
        grid=(NG,),
        in_specs=[
            pl.BlockSpec((D, TW), lambda g: (0, g)),
            pl.BlockSpec((D, TW), lambda g: (0, g)),
        ],
        out_specs=[
            pl.BlockSpec((Q, 128), lambda g: (g, 0)),
            pl.BlockSpec((Q, 128), lambda g: (g, 0)),
        ],
        out_shape=[
            jax.ShapeDtypeStruct((NG * Q, 128), jnp.float32),
            jax.ShapeDtypeStruct((NG * Q, 128), jnp.float32),
        ],
    )(ut_t, it_t)


_TW_SHIFT = TW.bit_length() - 1
_Q_SHIFT = Q.bit_length() - 1


def _packed_idx(r):
    return ((r >> _TW_SHIFT) * TW + ((r & (Q - 1)) << 2)
            + ((r >> _Q_SHIFT) & 3))


def _tc_loss(x2d):
    def body(x_ref, o_ref):
        x = x_ref[...]
        k = lax.iota(jnp.int32, 128)
        sel = (k[:, None] // L == lax.iota(jnp.int32, 8)[None, :])
        m = sel.astype(jnp.float32)
        d = jnp.dot(x, m, preferred_element_type=jnp.float32)
        sp = jnp.maximum(-d, 0.0) + jnp.log(1.0 + jnp.exp(-jnp.abs(d)))
        o_ref[0, 0] = jnp.sum(sp) * (1.0 / B)

    return pl.pallas_call(
        body,
        out_shape=jax.ShapeDtypeStruct((1, 1), jnp.float32),
        in_specs=[pl.BlockSpec(memory_space=pltpu.VMEM)],
        out_specs=pl.BlockSpec(memory_space=pltpu.SMEM),
    )(x2d)


def kernel(user_indices, pos_item_indices, neg_item_indices, user_table, item_table):
    u_idx = user_indices.astype(jnp.int32)
    p_idx = pos_item_indices.astype(jnp.int32)
    n_idx = neg_item_indices.astype(jnp.int32)
    ut_pack, it_pack = _tc_relayout(user_table.T, item_table.T)
    partials = _sc_partials(
        _packed_idx(u_idx), _packed_idx(p_idx), _packed_idx(n_idx),
        ut_pack.reshape(NP, D), it_pack.reshape(NP, D))
    loss = _tc_loss(partials.reshape(B * L // 128, 128))
    return loss[0, 0]
